# baseline (speedup 1.0000x reference)
import functools

import numpy as np
import jax
import jax.numpy as jnp
from jax import lax
from jax.experimental import pallas as pl
from jax.experimental.pallas import tpu as pltpu
from jax.experimental.pallas import tpu_sc as plsc

DIM = 128
B = 4096
S_TOK = 153
NGROUPS = 51
STRIDE = 1101
NID = 4000
RID = 4001

NW = 32
BPW = B // NW
NBUF = 6


def _pe_np(seq_len, dim):
    pos = np.arange(seq_len, dtype=np.float32)[:, None]
    div = np.exp(np.arange(0, dim, 2, dtype=np.float32) * (-np.log(10000.0) / dim))
    pe = np.zeros((seq_len, dim), dtype=np.float32)
    pe[:, 0::2] = np.sin(pos * div)
    pe[:, 1::2] = np.cos(pos * div)
    return pe


def _pe_groups():
    pe = _pe_np(S_TOK, DIM)
    rows = np.zeros((NGROUPS, 3, DIM), np.float32)
    for g in range(26):
        rows[g] = pe[[6 * g, 6 * g + 1, 6 * g + 2]]
    for g in range(25):
        rows[26 + g] = pe[[6 * g + 3, 6 * g + 4, 6 * g + 5]]
    return jnp.asarray(rows)


def _build_fused_body(emb_ref, type_ref, gamma_ref, beta_ref, pe_ref, out_ref):
    j = pl.program_id(0)
    is_node = j < 26
    sp = jnp.where(is_node, emb_ref[NID], emb_ref[RID])
    hi = jnp.where(is_node, emb_ref[1000:1100, :], emb_ref[3000:3100, :])
    lo = jnp.where(is_node, emb_ref[0:1000, :], emb_ref[2000:3000, :])
    t0, t1, t2 = type_ref[0], type_ref[1], type_ref[2]
    row_is0 = lax.broadcasted_iota(jnp.int32, (1000, 1), 0) == 0
    tlo = jnp.where(jnp.logical_and(is_node, row_is0), t0, t1)
    pe_sp, pe_hi, pe_lo = pe_ref[0, 0], pe_ref[0, 1], pe_ref[0, 2]
    x = jnp.concatenate(
        [
            (sp + pe_sp + t2)[None, :],
            hi + pe_hi[None, :] + t1[None, :],
            lo + pe_lo[None, :] + tlo,
        ],
        axis=0,
    )
    m = jnp.mean(x, axis=-1, keepdims=True)
    v = jnp.mean((x - m) ** 2, axis=-1, keepdims=True)
    y = (x - m) * lax.rsqrt(v + 1e-5)
    out_ref[0] = y * gamma_ref[...][None, :] + beta_ref[...][None, :]


def _build_fused(emb_table, type_table, gamma, beta):
    pe_grp = _pe_groups()
    return pl.pallas_call(
        _build_fused_body,
        grid=(NGROUPS,),
        in_specs=[
            pl.BlockSpec(emb_table.shape, lambda j: (0, 0)),
            pl.BlockSpec(type_table.shape, lambda j: (0, 0)),
            pl.BlockSpec((DIM,), lambda j: (0,)),
            pl.BlockSpec((DIM,), lambda j: (0,)),
            pl.BlockSpec((1, 3, DIM), lambda j: (j, 0, 0)),
        ],
        out_specs=pl.BlockSpec((1, STRIDE, DIM), lambda j: (j, 0, 0)),
        out_shape=jax.ShapeDtypeStruct((NGROUPS, STRIDE, DIM), jnp.float32),
    )(emb_table, type_table, gamma, beta, pe_grp)


def _sc_body(seqT_hbm, ft_hbm, out_hbm, hid_hbm, seq_v, fid_v, hid_v,
             bufs, gsems, ssems):
    wid = lax.axis_index("s") * 2 + lax.axis_index("c")
    b0 = wid * BPW

    pltpu.sync_copy(seqT_hbm.at[:, pl.ds(b0, BPW)], seq_v)

    def fill_row(col, irow, qrow, base):
        for j in range(BPW // 16):
            sl = pl.ds(16 * j, 16)
            vals = seq_v[col, sl]
            fid_v[irow, sl] = lax.rem(vals, 1000) + (base + 101)
            hid_v[qrow, sl] = lax.div(vals, 1000)

    def grp_body(g, carry):
        fill_row(2 * g, 2 * g, g, g * STRIDE)
        fill_row(2 * g + 1, 2 * g + 1, 26 + g, (26 + g) * STRIDE)
        return carry

    lax.fori_loop(0, 25, grp_body, 0)
    fill_row(50, 50, 25, 25 * STRIDE)
    pltpu.sync_copy(hid_v, hid_hbm.at[:, pl.ds(b0, BPW)])

    NLO = 51

    def g_issue(i, k):
        pltpu.async_copy(ft_hbm.at[fid_v.at[i]], bufs[k], gsems[k])

    def g_wait(i, k):
        pltpu.make_async_copy(ft_hbm.at[fid_v.at[i]], bufs[k], gsems[k]).wait()

    def s_issue(i, k):
        pltpu.async_copy(bufs[k], out_hbm.at[3 * i + 2, pl.ds(b0, BPW)],
                         ssems[k])

    def s_wait(i, k):
        pltpu.make_async_copy(
            bufs[k], out_hbm.at[3 * i + 2, pl.ds(b0, BPW)], ssems[k]).wait()

    def step(i, o, first_fill, refill):
        g_wait(i, o)
        s_issue(i, o)
        if refill:
            k5 = (o + NBUF - 1) % NBUF
            if not first_fill:
                s_wait(i - 1, k5)
            g_issue(i + NBUF - 1, k5)

    for o in range(NBUF - 1):
        g_issue(o, o)
    for o in range(NBUF):
        step(o, o, first_fill=(o == 0), refill=True)

    def pipe_body(t, carry):
        for o in range(NBUF):
            step(NBUF * t + o, o, first_fill=False, refill=True)
        return carry

    n_full = (NLO - (NBUF - 1)) // NBUF
    lax.fori_loop(1, n_full, pipe_body, 0)

    for i in range(NBUF * n_full, NLO):
        step(i, i % NBUF, first_fill=False, refill=(i + NBUF - 1 < NLO))

    for k in range(NBUF):
        s_wait(NLO - 1 - ((NLO - 1 - k) % NBUF), k)


@functools.partial(
    pl.kernel,
    out_type=(jax.ShapeDtypeStruct((S_TOK, B, DIM), jnp.float32),
              jax.ShapeDtypeStruct((51, B), jnp.int32)),
    mesh=plsc.VectorSubcoreMesh(core_axis_name="c", subcore_axis_name="s"),
    scratch_types=(
        [pltpu.VMEM((51, BPW), jnp.int32),
         pltpu.VMEM((51, BPW), jnp.int32),
         pltpu.VMEM((51, BPW), jnp.int32)]
        + [pltpu.VMEM((BPW, DIM), jnp.float32)] * NBUF
        + [pltpu.SemaphoreType.DMA] * (2 * NBUF)
    ),
)
def _sc_lookup(seqT_hbm, ft_hbm, out_hbm, hid_hbm, seq_v, fid_v, hid_v, *rest):
    bufs = rest[:NBUF]
    gsems = rest[NBUF:2 * NBUF]
    ssems = rest[2 * NBUF:]
    _sc_body(seqT_hbm, ft_hbm, out_hbm, hid_hbm, seq_v, fid_v, hid_v,
             bufs, gsems, ssems)


def _dense_body(ft_ref, hid_ref, _prev_ref, out_ref):
    p = pl.program_id(0)
    d_raw = hid_ref[0, 0, :]
    d_idx = jnp.where(p < NGROUPS, 0, d_raw + 1)
    sub = ft_ref[0, 0:104, :]
    oh16 = (lax.broadcasted_iota(jnp.int32, (B, 104), 1)
            == d_idx[:, None]).astype(jnp.bfloat16)
    s_hi = sub.astype(jnp.bfloat16)
    s_lo = (sub - s_hi.astype(jnp.float32)).astype(jnp.bfloat16)
    out_ref[0] = (jnp.dot(oh16, s_hi, preferred_element_type=jnp.float32)
                  + jnp.dot(oh16, s_lo, preferred_element_type=jnp.float32))


def _write_dense(ft3, hid3, out):
    def grp(p):
        return jnp.where(p < NGROUPS, p, p - NGROUPS)

    def row(p):
        return jnp.where(
            p < 26, 6 * p,
            jnp.where(p < 51, 6 * p - 153,
                      jnp.where(p < 77, 6 * p - 305, 6 * p - 458)))

    return pl.pallas_call(
        _dense_body,
        grid=(2 * NGROUPS,),
        in_specs=[
            pl.BlockSpec((1, 104, DIM), lambda p: (grp(p), 0, 0)),
            pl.BlockSpec((1, 1, B), lambda p: (grp(p), 0, 0)),
            pl.BlockSpec(memory_space=pl.ANY),
        ],
        out_specs=pl.BlockSpec((1, B, DIM), lambda p: (row(p), 0, 0)),
        out_shape=jax.ShapeDtypeStruct((S_TOK, B, DIM), jnp.float32),
        input_output_aliases={2: 0},
    )(ft3, hid3, out)


def kernel(seq, emb_table, type_table, gamma, beta):
    ft3 = _build_fused(emb_table, type_table, gamma, beta)
    ft = ft3.reshape(NGROUPS * STRIDE, DIM)
    out, hid = _sc_lookup(seq.T, ft)
    return _write_dense(ft3, hid.reshape(51, 1, B), out)


# speedup vs baseline: 1.0600x; 1.0600x over previous
import functools

import numpy as np
import jax
import jax.numpy as jnp
from jax import lax
from jax.experimental import pallas as pl
from jax.experimental.pallas import tpu as pltpu
from jax.experimental.pallas import tpu_sc as plsc

DIM = 128
B = 4096
S_TOK = 153
NGROUPS = 51
STRIDE = 1101
NID = 4000
RID = 4001

NW = 32
BPW = B // NW
NBUF = 7


def _pe_np(seq_len, dim):
    pos = np.arange(seq_len, dtype=np.float32)[:, None]
    div = np.exp(np.arange(0, dim, 2, dtype=np.float32) * (-np.log(10000.0) / dim))
    pe = np.zeros((seq_len, dim), dtype=np.float32)
    pe[:, 0::2] = np.sin(pos * div)
    pe[:, 1::2] = np.cos(pos * div)
    return pe


def _pe_groups():
    pe = _pe_np(S_TOK, DIM)
    rows = np.zeros((NGROUPS, 3, DIM), np.float32)
    for g in range(26):
        rows[g] = pe[[6 * g, 6 * g + 1, 6 * g + 2]]
    for g in range(25):
        rows[26 + g] = pe[[6 * g + 3, 6 * g + 4, 6 * g + 5]]
    return jnp.asarray(rows)


def _build_fused_body(emb_ref, type_ref, gamma_ref, beta_ref, pe_ref, out_ref):
    j = pl.program_id(0)
    is_node = j < 26
    sp = jnp.where(is_node, emb_ref[NID], emb_ref[RID])
    hi = jnp.where(is_node, emb_ref[1000:1100, :], emb_ref[3000:3100, :])
    lo = jnp.where(is_node, emb_ref[0:1000, :], emb_ref[2000:3000, :])
    t0, t1, t2 = type_ref[0], type_ref[1], type_ref[2]
    row_is0 = lax.broadcasted_iota(jnp.int32, (1000, 1), 0) == 0
    tlo = jnp.where(jnp.logical_and(is_node, row_is0), t0, t1)
    pe_sp, pe_hi, pe_lo = pe_ref[0, 0], pe_ref[0, 1], pe_ref[0, 2]
    x = jnp.concatenate(
        [
            (sp + pe_sp + t2)[None, :],
            hi + pe_hi[None, :] + t1[None, :],
            lo + pe_lo[None, :] + tlo,
        ],
        axis=0,
    )
    m = jnp.mean(x, axis=-1, keepdims=True)
    v = jnp.mean((x - m) ** 2, axis=-1, keepdims=True)
    y = (x - m) * lax.rsqrt(v + 1e-5)
    out_ref[0] = y * gamma_ref[...][None, :] + beta_ref[...][None, :]


def _build_fused(emb_table, type_table, gamma, beta):
    pe_grp = _pe_groups()
    return pl.pallas_call(
        _build_fused_body,
        grid=(NGROUPS,),
        in_specs=[
            pl.BlockSpec(emb_table.shape, lambda j: (0, 0)),
            pl.BlockSpec(type_table.shape, lambda j: (0, 0)),
            pl.BlockSpec((DIM,), lambda j: (0,)),
            pl.BlockSpec((DIM,), lambda j: (0,)),
            pl.BlockSpec((1, 3, DIM), lambda j: (j, 0, 0)),
        ],
        out_specs=pl.BlockSpec((1, STRIDE, DIM), lambda j: (j, 0, 0)),
        out_shape=jax.ShapeDtypeStruct((NGROUPS, STRIDE, DIM), jnp.float32),
    )(emb_table, type_table, gamma, beta, pe_grp)


def _sc_body(seqT_hbm, ft_hbm, out_hbm, seq_v, fid_v, bufs, gsems, ssems):
    wid = lax.axis_index("s") * 2 + lax.axis_index("c")
    b0 = wid * BPW
    iota = lax.iota(jnp.int32, 16)

    pltpu.sync_copy(seqT_hbm.at[:, pl.ds(b0, BPW)], seq_v)

    def fill_row(col, irow, base):
        for j in range(BPW // 16):
            sl = pl.ds(16 * j, 16)
            lo = lax.rem(seq_v[col, sl], 1000)
            fid_v[irow, sl] = lo + (base + 101)

    def grp_body(g, carry):
        fill_row(2 * g, 2 * g, g * STRIDE)
        fill_row(2 * g + 1, 2 * g + 1, (26 + g) * STRIDE)
        return carry

    lax.fori_loop(0, 25, grp_body, 0)
    fill_row(50, 50, 25 * STRIDE)

    NLO = 51

    def g_issue(i, k):
        pltpu.async_copy(ft_hbm.at[fid_v.at[i]], bufs[k], gsems[k])

    def g_wait(i, k):
        pltpu.make_async_copy(ft_hbm.at[fid_v.at[i]], bufs[k], gsems[k]).wait()

    def s_issue(i, k):
        pltpu.async_copy(bufs[k], out_hbm.at[3 * i + 2, pl.ds(b0, BPW)],
                         ssems[k])

    def s_wait(i, k):
        pltpu.make_async_copy(
            bufs[k], out_hbm.at[3 * i + 2, pl.ds(b0, BPW)], ssems[k]).wait()

    def step(i, o, first_fill, refill):
        g_wait(i, o)
        s_issue(i, o)
        if refill:
            k5 = (o + NBUF - 1) % NBUF
            if not first_fill:
                s_wait(i - 1, k5)
            g_issue(i + NBUF - 1, k5)

    for o in range(NBUF - 1):
        g_issue(o, o)
    for o in range(NBUF):
        step(o, o, first_fill=(o == 0), refill=True)

    def pipe_body(t, carry):
        for o in range(NBUF):
            step(NBUF * t + o, o, first_fill=False, refill=True)
        return carry

    n_full = (NLO - (NBUF - 1)) // NBUF
    lax.fori_loop(1, n_full, pipe_body, 0)

    for i in range(NBUF * n_full, NLO):
        step(i, i % NBUF, first_fill=False, refill=(i + NBUF - 1 < NLO))

    for k in range(NBUF):
        s_wait(NLO - 1 - ((NLO - 1 - k) % NBUF), k)


@functools.partial(
    pl.kernel,
    out_type=jax.ShapeDtypeStruct((S_TOK, B, DIM), jnp.float32),
    mesh=plsc.VectorSubcoreMesh(core_axis_name="c", subcore_axis_name="s"),
    scratch_types=(
        [pltpu.VMEM((51, BPW), jnp.int32), pltpu.VMEM((51, BPW), jnp.int32)]
        + [pltpu.VMEM((BPW, DIM), jnp.float32)] * NBUF
        + [pltpu.SemaphoreType.DMA] * (2 * NBUF)
    ),
)
def _sc_lookup(seqT_hbm, ft_hbm, out_hbm, seq_v, fid_v, *rest):
    bufs = rest[:NBUF]
    gsems = rest[NBUF:2 * NBUF]
    ssems = rest[2 * NBUF:]
    _sc_body(seqT_hbm, ft_hbm, out_hbm, seq_v, fid_v, bufs, gsems, ssems)


def _specials_body(ft_ref, _prev_ref, out_ref):
    out_ref[0] = jnp.broadcast_to(ft_ref[0, 0], (B, DIM))


def _write_specials(ft3, out):
    def sp_row(j):
        return jnp.where(j < 26, 6 * j, 6 * j - 153)

    return pl.pallas_call(
        _specials_body,
        grid=(NGROUPS,),
        in_specs=[
            pl.BlockSpec((1, 8, DIM), lambda j: (j, 0, 0)),
            pl.BlockSpec(memory_space=pl.ANY),
        ],
        out_specs=pl.BlockSpec((1, B, DIM), lambda j: (sp_row(j), 0, 0)),
        out_shape=jax.ShapeDtypeStruct((S_TOK, B, DIM), jnp.float32),
        input_output_aliases={1: 0},
    )(ft3, out)


def _hi_body(ft_ref, seq_ref, _prev_ref, out_ref):
    v = seq_ref[0, 0, :]
    hi = v // 1000
    sub = ft_ref[0, 1:101, :]
    oh = (lax.broadcasted_iota(jnp.int32, (B, 100), 1) == hi[:, None])
    oh16 = oh.astype(jnp.bfloat16)
    s_hi = sub.astype(jnp.bfloat16)
    s_lo = (sub - s_hi.astype(jnp.float32)).astype(jnp.bfloat16)
    out_ref[0] = (jnp.dot(oh16, s_hi, preferred_element_type=jnp.float32)
                  + jnp.dot(oh16, s_lo, preferred_element_type=jnp.float32))


def _write_high(ft3, seqT3, out):
    def seq_row(p):
        return jnp.where(p < 26, 2 * p, 2 * p - 51)

    def hi_row(p):
        return jnp.where(p < 26, 6 * p + 1, 6 * p - 152)

    return pl.pallas_call(
        _hi_body,
        grid=(NGROUPS,),
        in_specs=[
            pl.BlockSpec((1, 112, DIM), lambda p: (p, 0, 0)),
            pl.BlockSpec((1, 1, B), lambda p: (seq_row(p), 0, 0)),
            pl.BlockSpec(memory_space=pl.ANY),
        ],
        out_specs=pl.BlockSpec((1, B, DIM), lambda p: (hi_row(p), 0, 0)),
        out_shape=jax.ShapeDtypeStruct((S_TOK, B, DIM), jnp.float32),
        input_output_aliases={2: 0},
    )(ft3, seqT3, out)


def kernel(seq, emb_table, type_table, gamma, beta):
    ft3 = _build_fused(emb_table, type_table, gamma, beta)
    ft = ft3.reshape(NGROUPS * STRIDE, DIM)
    seqT = seq.T
    out = _sc_lookup(seqT, ft)
    out = _write_specials(ft3, out)
    return _write_high(ft3, seqT.reshape(51, 1, B), out)


# speedup vs baseline: 1.1867x; 1.1195x over previous
import functools

import numpy as np
import jax
import jax.numpy as jnp
from jax import lax
from jax.experimental import pallas as pl
from jax.experimental.pallas import tpu as pltpu
from jax.experimental.pallas import tpu_sc as plsc

DIM = 128
B = 4096
S_TOK = 153
NGROUPS = 51
STRIDE = 1104
NID = 4000
RID = 4001

NW = 32
BPW = B // NW
NBUF = 7


def _pe_np(seq_len, dim):
    pos = np.arange(seq_len, dtype=np.float32)[:, None]
    div = np.exp(np.arange(0, dim, 2, dtype=np.float32) * (-np.log(10000.0) / dim))
    pe = np.zeros((seq_len, dim), dtype=np.float32)
    pe[:, 0::2] = np.sin(pos * div)
    pe[:, 1::2] = np.cos(pos * div)
    return pe


def _pe_groups():
    pe = _pe_np(S_TOK, DIM)
    rows = np.zeros((NGROUPS, 3, DIM), np.float32)
    for g in range(26):
        rows[g] = pe[[6 * g, 6 * g + 1, 6 * g + 2]]
    for g in range(25):
        rows[26 + g] = pe[[6 * g + 3, 6 * g + 4, 6 * g + 5]]
    return jnp.asarray(rows)


def _build_fused_body(emb_ref, type_ref, gamma_ref, beta_ref, pe_ref, out_ref):
    j = pl.program_id(0)
    is_node = j < 26
    sp = jnp.where(is_node, emb_ref[NID], emb_ref[RID])
    hi = jnp.where(is_node, emb_ref[1000:1100, :], emb_ref[3000:3100, :])
    lo = jnp.where(is_node, emb_ref[0:1000, :], emb_ref[2000:3000, :])
    t0, t1, t2 = type_ref[0], type_ref[1], type_ref[2]
    row_is0 = lax.broadcasted_iota(jnp.int32, (1000, 1), 0) == 0
    tlo = jnp.where(jnp.logical_and(is_node, row_is0), t0, t1)
    pe_sp, pe_hi, pe_lo = pe_ref[0, 0], pe_ref[0, 1], pe_ref[0, 2]
    x = jnp.concatenate(
        [
            (sp + pe_sp + t2)[None, :],
            hi + pe_hi[None, :] + t1[None, :],
            lo + pe_lo[None, :] + tlo,
            jnp.zeros((3, DIM), jnp.float32),
        ],
        axis=0,
    )
    m = jnp.mean(x, axis=-1, keepdims=True)
    v = jnp.mean((x - m) ** 2, axis=-1, keepdims=True)
    y = (x - m) * lax.rsqrt(v + 1e-5)
    out_ref[0] = y * gamma_ref[...][None, :] + beta_ref[...][None, :]


def _build_fused(emb_table, type_table, gamma, beta):
    pe_grp = _pe_groups()
    return pl.pallas_call(
        _build_fused_body,
        grid=(NGROUPS,),
        in_specs=[
            pl.BlockSpec(emb_table.shape, lambda j: (0, 0)),
            pl.BlockSpec(type_table.shape, lambda j: (0, 0)),
            pl.BlockSpec((DIM,), lambda j: (0,)),
            pl.BlockSpec((DIM,), lambda j: (0,)),
            pl.BlockSpec((1, 3, DIM), lambda j: (j, 0, 0)),
        ],
        out_specs=pl.BlockSpec((1, STRIDE, DIM), lambda j: (j, 0, 0)),
        out_shape=jax.ShapeDtypeStruct((NGROUPS, STRIDE, DIM), jnp.float32),
    )(emb_table, type_table, gamma, beta, pe_grp)


def _sc_body(seqT_hbm, ft_hbm, out_hbm, seq_v, fid_v, bufs, gsems, ssems):
    wid = lax.axis_index("s") * 2 + lax.axis_index("c")
    b0 = wid * BPW
    iota = lax.iota(jnp.int32, 16)

    pltpu.sync_copy(seqT_hbm.at[:, pl.ds(b0, BPW)], seq_v)

    def fill_row(col, irow, base):
        for j in range(BPW // 16):
            sl = pl.ds(16 * j, 16)
            lo = lax.rem(seq_v[col, sl], 1000)
            fid_v[irow, sl] = lo + (base + 101)

    def grp_body(g, carry):
        fill_row(2 * g, 2 * g, g * STRIDE)
        fill_row(2 * g + 1, 2 * g + 1, (26 + g) * STRIDE)
        return carry

    lax.fori_loop(0, 25, grp_body, 0)
    fill_row(50, 50, 25 * STRIDE)

    NLO = 51

    def g_issue(i, k):
        pltpu.async_copy(ft_hbm.at[fid_v.at[i]], bufs[k], gsems[k])

    def g_wait(i, k):
        pltpu.make_async_copy(ft_hbm.at[fid_v.at[i]], bufs[k], gsems[k]).wait()

    def s_issue(i, k):
        pltpu.async_copy(bufs[k], out_hbm.at[3 * i + 2, pl.ds(b0, BPW)],
                         ssems[k])

    def s_wait(i, k):
        pltpu.make_async_copy(
            bufs[k], out_hbm.at[3 * i + 2, pl.ds(b0, BPW)], ssems[k]).wait()

    def step(i, o, first_fill, refill):
        g_wait(i, o)
        s_issue(i, o)
        if refill:
            k5 = (o + NBUF - 1) % NBUF
            if not first_fill:
                s_wait(i - 1, k5)
            g_issue(i + NBUF - 1, k5)

    for o in range(NBUF - 1):
        g_issue(o, o)
    for o in range(NBUF):
        step(o, o, first_fill=(o == 0), refill=True)

    def pipe_body(t, carry):
        for o in range(NBUF):
            step(NBUF * t + o, o, first_fill=False, refill=True)
        return carry

    n_full = (NLO - (NBUF - 1)) // NBUF
    lax.fori_loop(1, n_full, pipe_body, 0)

    for i in range(NBUF * n_full, NLO):
        step(i, i % NBUF, first_fill=False, refill=(i + NBUF - 1 < NLO))

    for k in range(NBUF):
        s_wait(NLO - 1 - ((NLO - 1 - k) % NBUF), k)


@functools.partial(
    pl.kernel,
    out_type=jax.ShapeDtypeStruct((S_TOK, B, DIM), jnp.float32),
    mesh=plsc.VectorSubcoreMesh(core_axis_name="c", subcore_axis_name="s"),
    scratch_types=(
        [pltpu.VMEM((51, BPW), jnp.int32), pltpu.VMEM((51, BPW), jnp.int32)]
        + [pltpu.VMEM((BPW, DIM), jnp.float32)] * NBUF
        + [pltpu.SemaphoreType.DMA] * (2 * NBUF)
    ),
)
def _sc_lookup(seqT_hbm, ft_hbm, out_hbm, seq_v, fid_v, *rest):
    bufs = rest[:NBUF]
    gsems = rest[NBUF:2 * NBUF]
    ssems = rest[2 * NBUF:]
    _sc_body(seqT_hbm, ft_hbm, out_hbm, seq_v, fid_v, bufs, gsems, ssems)


def _specials_body(ft_ref, _prev_ref, out_ref):
    out_ref[0] = jnp.broadcast_to(ft_ref[0, 0], (B, DIM))


def _write_specials(ft3, out):
    def sp_row(j):
        return jnp.where(j < 26, 6 * j, 6 * j - 153)

    return pl.pallas_call(
        _specials_body,
        grid=(NGROUPS,),
        in_specs=[
            pl.BlockSpec((1, 8, DIM), lambda j: (j, 0, 0)),
            pl.BlockSpec(memory_space=pl.ANY),
        ],
        out_specs=pl.BlockSpec((1, B, DIM), lambda j: (sp_row(j), 0, 0)),
        out_shape=jax.ShapeDtypeStruct((S_TOK, B, DIM), jnp.float32),
        input_output_aliases={1: 0},
    )(ft3, out)


def _hi_body(ft_ref, seq_ref, _prev_ref, out_ref):
    v = seq_ref[0, 0, :]
    hi = v // 1000
    sub = ft_ref[0, 1:101, :]
    oh = (lax.broadcasted_iota(jnp.int32, (B, 100), 1) == hi[:, None])
    oh16 = oh.astype(jnp.bfloat16)
    s_hi = sub.astype(jnp.bfloat16)
    s_lo = (sub - s_hi.astype(jnp.float32)).astype(jnp.bfloat16)
    out_ref[0] = (jnp.dot(oh16, s_hi, preferred_element_type=jnp.float32)
                  + jnp.dot(oh16, s_lo, preferred_element_type=jnp.float32))


def _write_high(ft3, seqT3, out):
    def seq_row(p):
        return jnp.where(p < 26, 2 * p, 2 * p - 51)

    def hi_row(p):
        return jnp.where(p < 26, 6 * p + 1, 6 * p - 152)

    return pl.pallas_call(
        _hi_body,
        grid=(NGROUPS,),
        in_specs=[
            pl.BlockSpec((1, 112, DIM), lambda p: (p, 0, 0)),
            pl.BlockSpec((1, 1, B), lambda p: (seq_row(p), 0, 0)),
            pl.BlockSpec(memory_space=pl.ANY),
        ],
        out_specs=pl.BlockSpec((1, B, DIM), lambda p: (hi_row(p), 0, 0)),
        out_shape=jax.ShapeDtypeStruct((S_TOK, B, DIM), jnp.float32),
        input_output_aliases={2: 0},
    )(ft3, seqT3, out)


def kernel(seq, emb_table, type_table, gamma, beta):
    ft3 = _build_fused(emb_table, type_table, gamma, beta)
    ft = ft3.reshape(NGROUPS * STRIDE, DIM)
    seqT = seq.T
    out = _sc_lookup(seqT, ft)
    out = _write_specials(ft3, out)
    return _write_high(ft3, seqT.reshape(51, 1, B), out)


# speedup vs baseline: 1.2536x; 1.0564x over previous
import functools

import numpy as np
import jax
import jax.numpy as jnp
from jax import lax
from jax.experimental import pallas as pl
from jax.experimental.pallas import tpu as pltpu
from jax.experimental.pallas import tpu_sc as plsc

DIM = 128
B = 4096
S_TOK = 153
NGROUPS = 51
STRIDE = 1104
NID = 4000
RID = 4001

NW = 32
BPW = B // NW
NBUF = 7


def _pe_np(seq_len, dim):
    pos = np.arange(seq_len, dtype=np.float32)[:, None]
    div = np.exp(np.arange(0, dim, 2, dtype=np.float32) * (-np.log(10000.0) / dim))
    pe = np.zeros((seq_len, dim), dtype=np.float32)
    pe[:, 0::2] = np.sin(pos * div)
    pe[:, 1::2] = np.cos(pos * div)
    return pe


def _pe_groups():
    pe = _pe_np(S_TOK, DIM)
    rows = np.zeros((NGROUPS, 3, DIM), np.float32)
    for g in range(26):
        rows[g] = pe[[6 * g, 6 * g + 1, 6 * g + 2]]
    for g in range(25):
        rows[26 + g] = pe[[6 * g + 3, 6 * g + 4, 6 * g + 5]]
    return jnp.asarray(rows)


def _build_fused_body(emb_ref, type_ref, gamma_ref, beta_ref, pe_ref, out_ref):
    j = pl.program_id(0)
    is_node = j < 26
    sp = jnp.where(is_node, emb_ref[NID], emb_ref[RID])
    hi = jnp.where(is_node, emb_ref[1000:1100, :], emb_ref[3000:3100, :])
    lo = jnp.where(is_node, emb_ref[0:1000, :], emb_ref[2000:3000, :])
    t0, t1, t2 = type_ref[0], type_ref[1], type_ref[2]
    row_is0 = lax.broadcasted_iota(jnp.int32, (1000, 1), 0) == 0
    tlo = jnp.where(jnp.logical_and(is_node, row_is0), t0, t1)
    pe_sp, pe_hi, pe_lo = pe_ref[0, 0], pe_ref[0, 1], pe_ref[0, 2]
    x = jnp.concatenate(
        [
            (sp + pe_sp + t2)[None, :],
            hi + pe_hi[None, :] + t1[None, :],
            lo + pe_lo[None, :] + tlo,
            jnp.zeros((3, DIM), jnp.float32),
        ],
        axis=0,
    )
    m = jnp.mean(x, axis=-1, keepdims=True)
    v = jnp.mean((x - m) ** 2, axis=-1, keepdims=True)
    y = (x - m) * lax.rsqrt(v + 1e-5)
    out_ref[0] = y * gamma_ref[...][None, :] + beta_ref[...][None, :]


def _build_fused(emb_table, type_table, gamma, beta):
    pe_grp = _pe_groups()
    return pl.pallas_call(
        _build_fused_body,
        grid=(NGROUPS,),
        in_specs=[
            pl.BlockSpec(emb_table.shape, lambda j: (0, 0)),
            pl.BlockSpec(type_table.shape, lambda j: (0, 0)),
            pl.BlockSpec((DIM,), lambda j: (0,)),
            pl.BlockSpec((DIM,), lambda j: (0,)),
            pl.BlockSpec((1, 3, DIM), lambda j: (j, 0, 0)),
        ],
        out_specs=pl.BlockSpec((1, STRIDE, DIM), lambda j: (j, 0, 0)),
        out_shape=jax.ShapeDtypeStruct((NGROUPS, STRIDE, DIM), jnp.float32),
    )(emb_table, type_table, gamma, beta, pe_grp)


def _sc_body(seqT_hbm, ft_hbm, out_hbm, seq_v, fid_v, bufs, gsems, ssems):
    wid = lax.axis_index("s") * 2 + lax.axis_index("c")
    b0 = wid * BPW
    iota = lax.iota(jnp.int32, 16)

    pltpu.sync_copy(seqT_hbm.at[:, pl.ds(b0, BPW)], seq_v)

    def fill_row(col, irow, base):
        for j in range(BPW // 16):
            sl = pl.ds(16 * j, 16)
            vals = seq_v[col, sl]
            q = (vals.astype(jnp.float32) * 0.001 + 0.5).astype(jnp.int32)
            r = vals - q * 1000
            lo = jnp.where(r < 0, r + 1000, r)
            fid_v[irow, sl] = lo + (base + 101)

    def grp_body(g, carry):
        fill_row(2 * g, 2 * g, g * STRIDE)
        fill_row(2 * g + 1, 2 * g + 1, (26 + g) * STRIDE)
        return carry

    lax.fori_loop(0, 25, grp_body, 0)
    fill_row(50, 50, 25 * STRIDE)

    NLO = 51

    def g_issue(i, k):
        pltpu.async_copy(ft_hbm.at[fid_v.at[i]], bufs[k], gsems[k])

    def g_wait(i, k):
        pltpu.make_async_copy(ft_hbm.at[fid_v.at[i]], bufs[k], gsems[k]).wait()

    def s_issue(i, k):
        pltpu.async_copy(bufs[k], out_hbm.at[3 * i + 2, pl.ds(b0, BPW)],
                         ssems[k])

    def s_wait(i, k):
        pltpu.make_async_copy(
            bufs[k], out_hbm.at[3 * i + 2, pl.ds(b0, BPW)], ssems[k]).wait()

    def step(i, o, first_fill, refill):
        g_wait(i, o)
        s_issue(i, o)
        if refill:
            k5 = (o + NBUF - 1) % NBUF
            if not first_fill:
                s_wait(i - 1, k5)
            g_issue(i + NBUF - 1, k5)

    for o in range(NBUF - 1):
        g_issue(o, o)
    for o in range(NBUF):
        step(o, o, first_fill=(o == 0), refill=True)

    def pipe_body(t, carry):
        for o in range(NBUF):
            step(NBUF * t + o, o, first_fill=False, refill=True)
        return carry

    n_full = (NLO - (NBUF - 1)) // NBUF
    lax.fori_loop(1, n_full, pipe_body, 0)

    for i in range(NBUF * n_full, NLO):
        step(i, i % NBUF, first_fill=False, refill=(i + NBUF - 1 < NLO))

    for k in range(NBUF):
        s_wait(NLO - 1 - ((NLO - 1 - k) % NBUF), k)


@functools.partial(
    pl.kernel,
    out_type=jax.ShapeDtypeStruct((S_TOK, B, DIM), jnp.float32),
    mesh=plsc.VectorSubcoreMesh(core_axis_name="c", subcore_axis_name="s"),
    scratch_types=(
        [pltpu.VMEM((51, BPW), jnp.int32), pltpu.VMEM((51, BPW), jnp.int32)]
        + [pltpu.VMEM((BPW, DIM), jnp.float32)] * NBUF
        + [pltpu.SemaphoreType.DMA] * (2 * NBUF)
    ),
)
def _sc_lookup(seqT_hbm, ft_hbm, out_hbm, seq_v, fid_v, *rest):
    bufs = rest[:NBUF]
    gsems = rest[NBUF:2 * NBUF]
    ssems = rest[2 * NBUF:]
    _sc_body(seqT_hbm, ft_hbm, out_hbm, seq_v, fid_v, bufs, gsems, ssems)


def _specials_body(ft_ref, _prev_ref, out_ref):
    out_ref[0] = jnp.broadcast_to(ft_ref[0, 0], (B, DIM))


def _write_specials(ft3, out):
    def sp_row(j):
        return jnp.where(j < 26, 6 * j, 6 * j - 153)

    return pl.pallas_call(
        _specials_body,
        grid=(NGROUPS,),
        in_specs=[
            pl.BlockSpec((1, 8, DIM), lambda j: (j, 0, 0)),
            pl.BlockSpec(memory_space=pl.ANY),
        ],
        out_specs=pl.BlockSpec((1, B, DIM), lambda j: (sp_row(j), 0, 0)),
        out_shape=jax.ShapeDtypeStruct((S_TOK, B, DIM), jnp.float32),
        input_output_aliases={1: 0},
    )(ft3, out)


def _hi_body(ft_ref, seq_ref, _prev_ref, out_ref):
    v = seq_ref[0, 0, :]
    hi = v // 1000
    sub = ft_ref[0, 1:101, :]
    oh = (lax.broadcasted_iota(jnp.int32, (B, 100), 1) == hi[:, None])
    oh16 = oh.astype(jnp.bfloat16)
    s_hi = sub.astype(jnp.bfloat16)
    s_lo = (sub - s_hi.astype(jnp.float32)).astype(jnp.bfloat16)
    out_ref[0] = (jnp.dot(oh16, s_hi, preferred_element_type=jnp.float32)
                  + jnp.dot(oh16, s_lo, preferred_element_type=jnp.float32))


def _write_high(ft3, seqT3, out):
    def seq_row(p):
        return jnp.where(p < 26, 2 * p, 2 * p - 51)

    def hi_row(p):
        return jnp.where(p < 26, 6 * p + 1, 6 * p - 152)

    return pl.pallas_call(
        _hi_body,
        grid=(NGROUPS,),
        in_specs=[
            pl.BlockSpec((1, 112, DIM), lambda p: (p, 0, 0)),
            pl.BlockSpec((1, 1, B), lambda p: (seq_row(p), 0, 0)),
            pl.BlockSpec(memory_space=pl.ANY),
        ],
        out_specs=pl.BlockSpec((1, B, DIM), lambda p: (hi_row(p), 0, 0)),
        out_shape=jax.ShapeDtypeStruct((S_TOK, B, DIM), jnp.float32),
        input_output_aliases={2: 0},
    )(ft3, seqT3, out)


def kernel(seq, emb_table, type_table, gamma, beta):
    ft3 = _build_fused(emb_table, type_table, gamma, beta)
    ft = ft3.reshape(NGROUPS * STRIDE, DIM)
    seqT = seq.T
    out = _sc_lookup(seqT, ft)
    out = _write_specials(ft3, out)
    return _write_high(ft3, seqT.reshape(51, 1, B), out)


# speedup vs baseline: 1.2555x; 1.0015x over previous
import functools

import numpy as np
import jax
import jax.numpy as jnp
from jax import lax
from jax.experimental import pallas as pl
from jax.experimental.pallas import tpu as pltpu
from jax.experimental.pallas import tpu_sc as plsc

DIM = 128
B = 4096
S_TOK = 153
NGROUPS = 51
STRIDE = 1104
NID = 4000
RID = 4001

NW = 32
BPW = B // NW
NBUF = 7


def _pe_np(seq_len, dim):
    pos = np.arange(seq_len, dtype=np.float32)[:, None]
    div = np.exp(np.arange(0, dim, 2, dtype=np.float32) * (-np.log(10000.0) / dim))
    pe = np.zeros((seq_len, dim), dtype=np.float32)
    pe[:, 0::2] = np.sin(pos * div)
    pe[:, 1::2] = np.cos(pos * div)
    return pe


def _pe_groups():
    pe = _pe_np(S_TOK, DIM)
    rows = np.zeros((NGROUPS, 3, DIM), np.float32)
    for g in range(26):
        rows[g] = pe[[6 * g, 6 * g + 1, 6 * g + 2]]
    for g in range(25):
        rows[26 + g] = pe[[6 * g + 3, 6 * g + 4, 6 * g + 5]]
    return jnp.asarray(rows)


def _build_fused_body(emb_ref, type_ref, gamma_ref, beta_ref, pe_ref, out_ref):
    j = pl.program_id(0)
    is_node = j < 26
    sp = jnp.where(is_node, emb_ref[NID], emb_ref[RID])
    hi = jnp.where(is_node, emb_ref[1000:1100, :], emb_ref[3000:3100, :])
    lo = jnp.where(is_node, emb_ref[0:1000, :], emb_ref[2000:3000, :])
    t0, t1, t2 = type_ref[0], type_ref[1], type_ref[2]
    row_is0 = lax.broadcasted_iota(jnp.int32, (1000, 1), 0) == 0
    tlo = jnp.where(jnp.logical_and(is_node, row_is0), t0, t1)
    pe_sp, pe_hi, pe_lo = pe_ref[0, 0], pe_ref[0, 1], pe_ref[0, 2]
    x = jnp.concatenate(
        [
            (sp + pe_sp + t2)[None, :],
            hi + pe_hi[None, :] + t1[None, :],
            lo + pe_lo[None, :] + tlo,
            jnp.zeros((3, DIM), jnp.float32),
        ],
        axis=0,
    )
    m = jnp.mean(x, axis=-1, keepdims=True)
    v = jnp.mean((x - m) ** 2, axis=-1, keepdims=True)
    y = (x - m) * lax.rsqrt(v + 1e-5)
    out_ref[0] = y * gamma_ref[...][None, :] + beta_ref[...][None, :]


def _build_fused(emb_table, type_table, gamma, beta):
    pe_grp = _pe_groups()
    return pl.pallas_call(
        _build_fused_body,
        grid=(NGROUPS,),
        in_specs=[
            pl.BlockSpec(emb_table.shape, lambda j: (0, 0)),
            pl.BlockSpec(type_table.shape, lambda j: (0, 0)),
            pl.BlockSpec((DIM,), lambda j: (0,)),
            pl.BlockSpec((DIM,), lambda j: (0,)),
            pl.BlockSpec((1, 3, DIM), lambda j: (j, 0, 0)),
        ],
        out_specs=pl.BlockSpec((1, STRIDE, DIM), lambda j: (j, 0, 0)),
        out_shape=jax.ShapeDtypeStruct((NGROUPS, STRIDE, DIM), jnp.float32),
    )(emb_table, type_table, gamma, beta, pe_grp)


def _sc_body(seqT_hbm, ft_hbm, out_hbm, seq_v, fid_v, bufs, gsems, ssems):
    wid = lax.axis_index("s") * 2 + lax.axis_index("c")
    b0 = wid * BPW
    iota = lax.iota(jnp.int32, 16)

    pltpu.sync_copy(seqT_hbm.at[:, pl.ds(b0, BPW)], seq_v)

    def fill_row(col, irow, base):
        for j in range(BPW // 16):
            sl = pl.ds(16 * j, 16)
            vals = seq_v[col, sl]
            q = (vals.astype(jnp.float32) * 0.001 + 0.5).astype(jnp.int32)
            r = vals - q * 1000
            lo = jnp.where(r < 0, r + 1000, r)
            fid_v[irow, sl] = lo + (base + 101)

    def grp_body(g, carry):
        fill_row(2 * g, 2 * g, g * STRIDE)
        fill_row(2 * g + 1, 2 * g + 1, (26 + g) * STRIDE)
        return carry

    lax.fori_loop(0, 25, grp_body, 0)
    fill_row(50, 50, 25 * STRIDE)

    NLO = 51

    def g_issue(i, k):
        pltpu.async_copy(ft_hbm.at[fid_v.at[i]], bufs[k], gsems[k])

    def g_wait(i, k):
        pltpu.make_async_copy(ft_hbm.at[fid_v.at[i]], bufs[k], gsems[k]).wait()

    def s_issue(i, k):
        pltpu.async_copy(bufs[k], out_hbm.at[3 * i + 2, pl.ds(b0, BPW)],
                         ssems[k])

    def s_wait(i, k):
        pltpu.make_async_copy(
            bufs[k], out_hbm.at[3 * i + 2, pl.ds(b0, BPW)], ssems[k]).wait()

    def step(i, o, first_fill, refill):
        g_wait(i, o)
        s_issue(i, o)
        if refill:
            k5 = (o + NBUF - 1) % NBUF
            if not first_fill:
                s_wait(i - 1, k5)
            g_issue(i + NBUF - 1, k5)

    for o in range(NBUF - 1):
        g_issue(o, o)
    for o in range(NBUF):
        step(o, o, first_fill=(o == 0), refill=True)

    def pipe_body(t, carry):
        for o in range(NBUF):
            step(NBUF * t + o, o, first_fill=False, refill=True)
        return carry

    n_full = (NLO - (NBUF - 1)) // NBUF
    lax.fori_loop(1, n_full, pipe_body, 0)

    for i in range(NBUF * n_full, NLO):
        step(i, i % NBUF, first_fill=False, refill=(i + NBUF - 1 < NLO))

    for k in range(NBUF):
        s_wait(NLO - 1 - ((NLO - 1 - k) % NBUF), k)


@functools.partial(
    pl.kernel,
    out_type=jax.ShapeDtypeStruct((S_TOK, B, DIM), jnp.float32),
    mesh=plsc.VectorSubcoreMesh(core_axis_name="c", subcore_axis_name="s"),
    scratch_types=(
        [pltpu.VMEM((51, BPW), jnp.int32), pltpu.VMEM((51, BPW), jnp.int32)]
        + [pltpu.VMEM((BPW, DIM), jnp.float32)] * NBUF
        + [pltpu.SemaphoreType.DMA] * (2 * NBUF)
    ),
)
def _sc_lookup(seqT_hbm, ft_hbm, out_hbm, seq_v, fid_v, *rest):
    bufs = rest[:NBUF]
    gsems = rest[NBUF:2 * NBUF]
    ssems = rest[2 * NBUF:]
    _sc_body(seqT_hbm, ft_hbm, out_hbm, seq_v, fid_v, bufs, gsems, ssems)


def _specials_body(ft_ref, _prev_ref, out_ref):
    out_ref[0] = jnp.broadcast_to(ft_ref[0, 0], (B, DIM))


def _write_specials(ft3, out):
    def sp_row(j):
        return jnp.where(j < 26, 6 * j, 6 * j - 153)

    return pl.pallas_call(
        _specials_body,
        grid=(NGROUPS,),
        in_specs=[
            pl.BlockSpec((1, 8, DIM), lambda j: (j, 0, 0)),
            pl.BlockSpec(memory_space=pl.ANY),
        ],
        out_specs=pl.BlockSpec((1, B, DIM), lambda j: (sp_row(j), 0, 0)),
        out_shape=jax.ShapeDtypeStruct((S_TOK, B, DIM), jnp.float32),
        input_output_aliases={1: 0},
    )(ft3, out)


def _hi_body(ft_ref, seq_ref, _prev_ref, out_ref):
    v = seq_ref[0, 0, :]
    hi = v // 1000
    sub = ft_ref[0, 1:101, :]
    oh = (lax.broadcasted_iota(jnp.int32, (B, 100), 1) == hi[:, None])
    oh16 = oh.astype(jnp.bfloat16)
    s_hi = sub.astype(jnp.bfloat16)
    out_ref[0] = jnp.dot(oh16, s_hi, preferred_element_type=jnp.float32)


def _write_high(ft3, seqT3, out):
    def seq_row(p):
        return jnp.where(p < 26, 2 * p, 2 * p - 51)

    def hi_row(p):
        return jnp.where(p < 26, 6 * p + 1, 6 * p - 152)

    return pl.pallas_call(
        _hi_body,
        grid=(NGROUPS,),
        in_specs=[
            pl.BlockSpec((1, 112, DIM), lambda p: (p, 0, 0)),
            pl.BlockSpec((1, 1, B), lambda p: (seq_row(p), 0, 0)),
            pl.BlockSpec(memory_space=pl.ANY),
        ],
        out_specs=pl.BlockSpec((1, B, DIM), lambda p: (hi_row(p), 0, 0)),
        out_shape=jax.ShapeDtypeStruct((S_TOK, B, DIM), jnp.float32),
        input_output_aliases={2: 0},
    )(ft3, seqT3, out)


def kernel(seq, emb_table, type_table, gamma, beta):
    ft3 = _build_fused(emb_table, type_table, gamma, beta)
    ft = ft3.reshape(NGROUPS * STRIDE, DIM)
    seqT = seq.T
    out = _sc_lookup(seqT, ft)
    out = _write_specials(ft3, out)
    return _write_high(ft3, seqT.reshape(51, 1, B), out)
